# baseline (device time: 31167 ns/iter reference)
import jax
import jax.numpy as jnp
from jax import lax
from jax.experimental import pallas as pl
from jax.experimental.pallas import tpu as pltpu

N_Y = 2
KQ = 8
KH = KQ // 2


def kernel(x):
    m_per, n = x.shape
    q_rows = m_per // 4
    rows = q_rows // KQ

    def body(x_ref, out_ref,
             y_send, y_recv, xf_send, xf_recv, zf_send, zf_recv,
             xd_send, xd_recv, zd_send, zd_recv, copy_sem):
        mx = lax.axis_index("x")
        my = lax.axis_index("y")
        mz = lax.axis_index("z")
        mp = lax.rem(mz, 2)
        y_peer = (mx, 1 - my, mz)
        x_peer = (1 - mx, my, mz)
        z_peer = (mx, my, mz + 1 - 2 * mp)

        barrier_sem = pltpu.get_barrier_semaphore()
        for nbr in (y_peer, x_peer, z_peer):
            pl.semaphore_signal(
                barrier_sem, inc=1,
                device_id=nbr, device_id_type=pl.DeviceIdType.MESH,
            )
        pl.semaphore_wait(barrier_sem, 3)

        out_base = my * m_per
        in_base = (1 - my) * m_per

        q_d = 2 * mx + mp
        q_x = 2 * (1 - mx) + mp
        q_z = 2 * mx + (1 - mp)
        q_g = 2 * (1 - mx) + (1 - mp)

        def rc(send_to, src, dst, ssem, rsem):
            return pltpu.make_async_remote_copy(
                src_ref=src, dst_ref=dst, send_sem=ssem, recv_sem=rsem,
                device_id=send_to, device_id_type=pl.DeviceIdType.MESH,
            )

        sends = []

        for c in range(KQ):
            off = q_d * q_rows + c * rows
            s = rc(y_peer,
                   x_ref.at[pl.ds(off, rows)],
                   out_ref.at[pl.ds(out_base + off, rows)],
                   y_send.at[c], y_recv.at[c])
            s.start()
            sends.append(s)

        local_copy = pltpu.make_async_copy(
            x_ref, out_ref.at[pl.ds(out_base, m_per)], copy_sem,
        )
        local_copy.start()

        for c in range(KQ):
            off = in_base + q_d * q_rows + c * rows
            rc(y_peer, x_ref.at[pl.ds(c * rows, rows)],
               out_ref.at[pl.ds(off, rows)],
               y_send.at[c], y_recv.at[c]).wait_recv()
            s = rc(x_peer,
                   out_ref.at[pl.ds(off, rows)],
                   out_ref.at[pl.ds(off, rows)],
                   xf_send.at[c], xf_recv.at[c])
            s.start()
            sends.append(s)
            s = rc(z_peer,
                   out_ref.at[pl.ds(off, rows)],
                   out_ref.at[pl.ds(off, rows)],
                   zf_send.at[c], zf_recv.at[c])
            s.start()
            sends.append(s)

        for c in range(KH):
            off = in_base + q_z * q_rows + c * rows
            rc(z_peer, x_ref.at[pl.ds(c * rows, rows)],
               out_ref.at[pl.ds(off, rows)],
               zf_send.at[c], zf_recv.at[c]).wait_recv()
            s = rc(x_peer,
                   out_ref.at[pl.ds(off, rows)],
                   out_ref.at[pl.ds(off, rows)],
                   xd_send.at[c], xd_recv.at[c])
            s.start()
            sends.append(s)

        for c in range(KH):
            off = in_base + q_x * q_rows + (KH + c) * rows
            rc(x_peer, x_ref.at[pl.ds(c * rows, rows)],
               out_ref.at[pl.ds(off, rows)],
               xf_send.at[KH + c], xf_recv.at[KH + c]).wait_recv()
            s = rc(z_peer,
                   out_ref.at[pl.ds(off, rows)],
                   out_ref.at[pl.ds(off, rows)],
                   zd_send.at[c], zd_recv.at[c])
            s.start()
            sends.append(s)

        for c in range(KH):
            off = in_base + q_x * q_rows + c * rows
            rc(x_peer, x_ref.at[pl.ds(c * rows, rows)],
               out_ref.at[pl.ds(off, rows)],
               xf_send.at[c], xf_recv.at[c]).wait_recv()
        for c in range(KH):
            off = in_base + q_z * q_rows + (KH + c) * rows
            rc(z_peer, x_ref.at[pl.ds(c * rows, rows)],
               out_ref.at[pl.ds(off, rows)],
               zf_send.at[KH + c], zf_recv.at[KH + c]).wait_recv()
        for c in range(KH):
            off = in_base + q_g * q_rows + c * rows
            rc(x_peer, x_ref.at[pl.ds(c * rows, rows)],
               out_ref.at[pl.ds(off, rows)],
               xd_send.at[c], xd_recv.at[c]).wait_recv()
        for c in range(KH):
            off = in_base + q_g * q_rows + (KH + c) * rows
            rc(z_peer, x_ref.at[pl.ds(c * rows, rows)],
               out_ref.at[pl.ds(off, rows)],
               zd_send.at[c], zd_recv.at[c]).wait_recv()

        for s in sends:
            s.wait_send()
        local_copy.wait()

    return pl.pallas_call(
        body,
        out_shape=jax.ShapeDtypeStruct((N_Y * m_per, n), x.dtype),
        in_specs=[pl.BlockSpec(memory_space=pltpu.VMEM)],
        out_specs=pl.BlockSpec(memory_space=pltpu.VMEM),
        scratch_shapes=[
            pltpu.SemaphoreType.DMA((KQ,)),
            pltpu.SemaphoreType.DMA((KQ,)),
            pltpu.SemaphoreType.DMA((KQ,)),
            pltpu.SemaphoreType.DMA((KQ,)),
            pltpu.SemaphoreType.DMA((KQ,)),
            pltpu.SemaphoreType.DMA((KQ,)),
            pltpu.SemaphoreType.DMA((KH,)),
            pltpu.SemaphoreType.DMA((KH,)),
            pltpu.SemaphoreType.DMA((KH,)),
            pltpu.SemaphoreType.DMA((KH,)),
            pltpu.SemaphoreType.DMA,
        ],
        compiler_params=pltpu.CompilerParams(collective_id=0),
    )(x)
